# Initial kernel scaffold; baseline (speedup 1.0000x reference)
import jax
import jax.numpy as jnp
from jax.experimental import pallas as pl


def kernel(x, edge_index, edge_weight, w_t, b_t, gamma, beta, w_g, b_g):
    raise NotImplementedError("write your pallas kernel here")


# speedup vs baseline: 4.8343x; 4.8343x over previous
import functools

import jax
import jax.numpy as jnp
from jax import lax
from jax.experimental import pallas as pl
from jax.experimental.pallas import tpu as pltpu
from jax.experimental.pallas import tpu_sc as plsc

T = 12
N = 10000
E = 320000
C = 128

NPAD = 10240
NC, NS, L = 2, 16, 16
NW = NC * NS
B = 128
EPT = 10112
NCH = EPT // B
EPAD = EPT * NW
ROWS_PT = NPAD // NS
HI = 10.0


def _softmax_body(ew_ref, out_ref):
    w = ew_ref[...]
    m = jnp.max(w)
    e = jnp.exp(w - m)
    out_ref[...] = e / jnp.sum(e)


def _softmax(ew):
    ew2 = ew.reshape(E // C, C)
    out = pl.pallas_call(
        _softmax_body,
        out_shape=jax.ShapeDtypeStruct((E // C, C), jnp.float32),
    )(ew2)
    return out.reshape(E)


DW = 16


def _deg_body(dst3_hbm, ewf_hbm, out_hbm, dst2, ewc, rows, acc, sem):
    cid = lax.axis_index("c")
    sid = lax.axis_index("s")
    wid = cid * NS + sid

    pltpu.sync_copy(dst3_hbm.at[wid], dst2)
    pltpu.sync_copy(ewf_hbm.at[pl.ds(wid * EPT, EPT)], ewc)

    if True:
        def zrow(i, _):
            rows[i] = jnp.zeros((DW,), jnp.float32)
            return _
        lax.fori_loop(0, B, zrow, 0)

        def z(i, _):
            pltpu.sync_copy(rows, acc.at[pl.ds(sid * ROWS_PT + i * B, B)])
            return _
        lax.fori_loop(0, ROWS_PT // B, z, 0)
        plsc.subcore_barrier()

        def chunk(j, _):
            def fill(b, _):
                wv = plsc.load_gather(
                    ewc, [jnp.full((L,), j * B + b, jnp.int32)])
                rows[b] = wv
                return _
            lax.fori_loop(0, B, fill, 0)
            pltpu.sync_copy(rows, acc.at[dst2.at[j]], add=True)
            return _
        lax.fori_loop(0, NCH, chunk, 0)
        plsc.subcore_barrier()
        pltpu.sync_copy(acc.at[pl.ds(sid * ROWS_PT, ROWS_PT)],
                        out_hbm.at[cid, pl.ds(sid * ROWS_PT, ROWS_PT)])


def _deg(dst3, ewn_p):
    mesh = plsc.VectorSubcoreMesh(
        core_axis_name="c", subcore_axis_name="s", num_cores=NC, num_subcores=NS)
    return pl.kernel(
        _deg_body,
        out_type=jax.ShapeDtypeStruct((NC, NPAD, DW), jnp.float32),
        mesh=mesh,
        compiler_params=pltpu.CompilerParams(needs_layout_passes=False),
        scratch_types=[
            pltpu.VMEM((NCH, B), jnp.int32),
            pltpu.VMEM((EPT,), jnp.float32),
            pltpu.VMEM((B, DW), jnp.float32),
            pltpu.VMEM_SHARED((NPAD, DW), jnp.float32),
            pltpu.SemaphoreType.DMA,
        ],
    )(dst3, ewn_p)


def _dis_body(degp_ref, out_ref):
    deg = degp_ref[0, :, 0] + degp_ref[1, :, 0]
    out_ref[...] = lax.rsqrt(deg.reshape(NPAD // C, C) + 1.0)


def _dis(degp):
    out = pl.pallas_call(
        _dis_body,
        out_shape=jax.ShapeDtypeStruct((NPAD // C, C), jnp.float32),
    )(degp)
    return out.reshape(NPAD)


NB1 = 256


def _stage1_body(x_ref, wt_ref, bt_ref, g_ref, b_ref, out_ref):
    xb = x_ref[...]
    W = wt_ref[...]
    xf = xb.reshape(T * NB1, C)
    dn = (((1,), (1,)), ((), ()))
    hp = jax.lax.Precision.HIGHEST
    y1 = lax.dot_general(xf, W[:, :, 1], dn, precision=hp,
                         preferred_element_type=jnp.float32).reshape(T, NB1, C)
    z0 = lax.dot_general(xf, W[:, :, 0], dn, precision=hp,
                         preferred_element_type=jnp.float32).reshape(T, NB1, C)
    z2 = lax.dot_general(xf, W[:, :, 2], dn, precision=hp,
                         preferred_element_type=jnp.float32).reshape(T, NB1, C)
    zpad = jnp.zeros((1, NB1, C), jnp.float32)
    y = y1 + jnp.concatenate([zpad, z0[:-1]], 0) \
           + jnp.concatenate([z2[1:], zpad], 0)
    y = y + bt_ref[...][None, None, :]
    y = jnp.clip(y, -HI, HI)
    mean = jnp.mean(y, axis=0, keepdims=True)
    var = jnp.mean((y - mean) ** 2, axis=0, keepdims=True)
    y = (y - mean) * lax.rsqrt(var + 1e-5)
    y = g_ref[...][None, None, :] * y + b_ref[...][None, None, :]
    out_ref[...] = jnp.maximum(y, 0.0)


def _stage1(x_pad, w_t, b_t, gamma, beta):
    grid = (NPAD // NB1,)
    return pl.pallas_call(
        _stage1_body,
        grid=grid,
        in_specs=[
            pl.BlockSpec((T, NB1, C), lambda i: (0, i, 0)),
            pl.BlockSpec((C, C, 3), lambda i: (0, 0, 0)),
            pl.BlockSpec((C,), lambda i: (0,)),
            pl.BlockSpec((C,), lambda i: (0,)),
            pl.BlockSpec((C,), lambda i: (0,)),
        ],
        out_specs=pl.BlockSpec((T, NB1, C), lambda i: (0, i, 0)),
        out_shape=jax.ShapeDtypeStruct((T, NPAD, C), jnp.float32),
    )(x_pad, w_t, b_t, gamma, beta)


def _main_body(table_hbm, epack_hbm, dis_hbm, zeros_hbm, out_hbm,
               ebuf, idxv, nbuf, disv, rows, acc, sem):
    cid = lax.axis_index("c")
    sid = lax.axis_index("s")
    wid = cid * NS + sid

    pltpu.sync_copy(dis_hbm, disv)

    def acc_zero():
        def z(i, _):
            pltpu.sync_copy(zeros_hbm,
                            acc.at[pl.ds(sid * ROWS_PT + i * B, B)])
            return _
        lax.fori_loop(0, ROWS_PT // B, z, 0)

    acc_zero()
    plsc.subcore_barrier()

    def t_step(t, _):
        toff = t * NPAD

        def chunk(j, _):
            pltpu.sync_copy(epack_hbm.at[wid, j], ebuf)

            def grp(g, _):
                sl = pl.ds(g * L, L)
                s16 = ebuf[0, sl]
                idxv[sl] = s16 + toff
                d16 = ebuf[1, sl]
                w16 = plsc.bitcast(ebuf[2, sl], jnp.float32)
                ds_ = plsc.load_gather(disv, [s16])
                dd_ = plsc.load_gather(disv, [d16])
                nbuf[sl] = ds_ * w16 * dd_
                return _
            lax.fori_loop(0, B // L, grp, 0)

            pltpu.async_copy(table_hbm.at[idxv], rows, sem).wait()

            def scale(b, _):
                nb = plsc.load_gather(nbuf, [jnp.full((L,), b, jnp.int32)])
                for k in range(C // L):
                    rows[b, pl.ds(k * L, L)] = rows[b, pl.ds(k * L, L)] * nb
                return _
            lax.fori_loop(0, B, scale, 0)

            pltpu.sync_copy(rows, acc.at[ebuf.at[1]], add=True)
            return _
        lax.fori_loop(0, NCH, chunk, 0)
        plsc.subcore_barrier()

        pltpu.sync_copy(acc.at[pl.ds(sid * ROWS_PT, ROWS_PT)],
                        out_hbm.at[cid, t, pl.ds(sid * ROWS_PT, ROWS_PT)])
        acc_zero()
        plsc.subcore_barrier()
        return _
    lax.fori_loop(0, T, t_step, 0)


def _main(table, epack, dis, zeros):
    mesh = plsc.VectorSubcoreMesh(
        core_axis_name="c", subcore_axis_name="s", num_cores=NC, num_subcores=NS)
    return pl.kernel(
        _main_body,
        out_type=jax.ShapeDtypeStruct((NC, T, NPAD, C), jnp.float32),
        mesh=mesh,
        compiler_params=pltpu.CompilerParams(needs_layout_passes=False),
        scratch_types=[
            pltpu.VMEM((3, B), jnp.int32),
            pltpu.VMEM((B,), jnp.int32),
            pltpu.VMEM((B,), jnp.float32),
            pltpu.VMEM((NPAD,), jnp.float32),
            pltpu.VMEM((B, C), jnp.float32),
            pltpu.VMEM_SHARED((NPAD, C), jnp.float32),
            pltpu.SemaphoreType.DMA,
        ],
    )(table, epack, dis, zeros)


NB6 = 512


def _out_body(agg_ref, xc_ref, dis_ref, wg_ref, bg_ref, out_ref):
    a = agg_ref[0, 0] + agg_ref[1, 0]
    d = dis_ref[0, 0]
    xcb = xc_ref[0]
    full = a + (d * d)[:, None] * xcb
    hp = jax.lax.Precision.HIGHEST
    z = lax.dot_general(full, wg_ref[...], (((1,), (1,)), ((), ())),
                        precision=hp, preferred_element_type=jnp.float32)
    z = z + bg_ref[...][None, :]
    z = jnp.maximum(z, 0.0)
    out_ref[0] = jnp.minimum(z, HI)


def _out(agg, xc, dis, w_g, b_g):
    grid = (T, NPAD // NB6)
    return pl.pallas_call(
        _out_body,
        grid=grid,
        in_specs=[
            pl.BlockSpec((NC, 1, NB6, C), lambda t, i: (0, t, i, 0)),
            pl.BlockSpec((1, NB6, C), lambda t, i: (t, i, 0)),
            pl.BlockSpec((1, 1, NB6), lambda t, i: (i, 0, 0)),
            pl.BlockSpec((C, C), lambda t, i: (0, 0)),
            pl.BlockSpec((C,), lambda t, i: (0,)),
        ],
        out_specs=pl.BlockSpec((1, NB6, C), lambda t, i: (t, i, 0)),
        out_shape=jax.ShapeDtypeStruct((T, NPAD, C), jnp.float32),
    )(agg, xc, dis.reshape(NPAD // NB6, 1, NB6), w_g, b_g)


def kernel(x, edge_index, edge_weight, w_t, b_t, gamma, beta, w_g, b_g):
    x_pad = jnp.pad(x, ((0, 0), (0, NPAD - N), (0, 0)))
    src = edge_index[0]
    dst = edge_index[1]
    ew_n = _softmax(edge_weight)
    src_p = jnp.pad(src, (0, EPAD - E))
    dst_p = jnp.pad(dst, (0, EPAD - E))
    ewn_p = jnp.pad(ew_n, (0, EPAD - E))

    dst3 = dst_p.reshape(NW, NCH, B)
    degp = _deg(dst3, ewn_p)
    dis = _dis(degp)
    xc = _stage1(x_pad, w_t, b_t, gamma, beta)

    table = xc.reshape(T * NPAD, C)
    src3 = src_p.reshape(NW, NCH, B)
    ew3 = lax.bitcast_convert_type(ewn_p, jnp.int32).reshape(NW, NCH, B)
    epack = jnp.stack([src3, dst3, ew3], axis=2)
    zeros = jnp.zeros((B, C), jnp.float32)
    agg = _main(table, epack, dis, zeros)

    out = _out(agg, xc, dis, w_g, b_g)
    return out[:, :N, :]


# speedup vs baseline: 6.4785x; 1.3401x over previous
import functools

import jax
import jax.numpy as jnp
from jax import lax
from jax.experimental import pallas as pl
from jax.experimental.pallas import tpu as pltpu
from jax.experimental.pallas import tpu_sc as plsc

T = 12
N = 10000
E = 320000
C = 128

NPAD = 10240
NC, NS, L = 2, 16, 16
NW = NC * NS
B = 128
EPT = 10112
NCH = EPT // B
EPAD = EPT * NW
ROWS_PT = NPAD // NS
HI = 10.0


def _softmax_body(ew_ref, out_ref):
    w = ew_ref[...]
    m = jnp.max(w)
    e = jnp.exp(w - m)
    out_ref[...] = e / jnp.sum(e)


def _softmax(ew):
    ew2 = ew.reshape(E // C, C)
    out = pl.pallas_call(
        _softmax_body,
        out_shape=jax.ShapeDtypeStruct((E // C, C), jnp.float32),
    )(ew2)
    return out.reshape(E)


DW = 16


def _deg_body(dst3_hbm, ewf_hbm, out_hbm, dst2, ewc, rows, acc, sem):
    cid = lax.axis_index("c")
    sid = lax.axis_index("s")
    wid = cid * NS + sid

    pltpu.sync_copy(dst3_hbm.at[wid], dst2)
    pltpu.sync_copy(ewf_hbm.at[pl.ds(wid * EPT, EPT)], ewc)

    if True:
        def zrow(i, _):
            rows[i] = jnp.zeros((DW,), jnp.float32)
            return _
        lax.fori_loop(0, B, zrow, 0)

        def z(i, _):
            pltpu.sync_copy(rows, acc.at[pl.ds(sid * ROWS_PT + i * B, B)])
            return _
        lax.fori_loop(0, ROWS_PT // B, z, 0)
        plsc.subcore_barrier()

        def chunk(j, _):
            def fill(b, _):
                wv = plsc.load_gather(
                    ewc, [jnp.full((L,), j * B + b, jnp.int32)])
                rows[b] = wv
                return _
            lax.fori_loop(0, B, fill, 0)
            pltpu.sync_copy(rows, acc.at[dst2.at[j]], add=True)
            return _
        lax.fori_loop(0, NCH, chunk, 0)
        plsc.subcore_barrier()
        pltpu.sync_copy(acc.at[pl.ds(sid * ROWS_PT, ROWS_PT)],
                        out_hbm.at[cid, pl.ds(sid * ROWS_PT, ROWS_PT)])


def _deg(dst3, ewn_p):
    mesh = plsc.VectorSubcoreMesh(
        core_axis_name="c", subcore_axis_name="s", num_cores=NC, num_subcores=NS)
    return pl.kernel(
        _deg_body,
        out_type=jax.ShapeDtypeStruct((NC, NPAD, DW), jnp.float32),
        mesh=mesh,
        compiler_params=pltpu.CompilerParams(needs_layout_passes=False),
        scratch_types=[
            pltpu.VMEM((NCH, B), jnp.int32),
            pltpu.VMEM((EPT,), jnp.float32),
            pltpu.VMEM((B, DW), jnp.float32),
            pltpu.VMEM_SHARED((NPAD, DW), jnp.float32),
            pltpu.SemaphoreType.DMA,
        ],
    )(dst3, ewn_p)


def _dis_body(degp_ref, out_ref):
    deg = degp_ref[0, :, 0] + degp_ref[1, :, 0]
    out_ref[...] = lax.rsqrt(deg.reshape(NPAD // C, C) + 1.0)


def _dis(degp):
    out = pl.pallas_call(
        _dis_body,
        out_shape=jax.ShapeDtypeStruct((NPAD // C, C), jnp.float32),
    )(degp)
    return out.reshape(NPAD)


NB1 = 256


def _stage1_body(x_ref, wt_ref, bt_ref, g_ref, b_ref, dis_ref, out_ref):
    xb = x_ref[...]
    W = wt_ref[...]
    xf = xb.reshape(T * NB1, C)
    dn = (((1,), (1,)), ((), ()))
    hp = jax.lax.Precision.HIGHEST
    y1 = lax.dot_general(xf, W[:, :, 1], dn, precision=hp,
                         preferred_element_type=jnp.float32).reshape(T, NB1, C)
    z0 = lax.dot_general(xf, W[:, :, 0], dn, precision=hp,
                         preferred_element_type=jnp.float32).reshape(T, NB1, C)
    z2 = lax.dot_general(xf, W[:, :, 2], dn, precision=hp,
                         preferred_element_type=jnp.float32).reshape(T, NB1, C)
    zpad = jnp.zeros((1, NB1, C), jnp.float32)
    y = y1 + jnp.concatenate([zpad, z0[:-1]], 0) \
           + jnp.concatenate([z2[1:], zpad], 0)
    y = y + bt_ref[...][None, None, :]
    y = jnp.clip(y, -HI, HI)
    mean = jnp.mean(y, axis=0, keepdims=True)
    var = jnp.mean((y - mean) ** 2, axis=0, keepdims=True)
    y = (y - mean) * lax.rsqrt(var + 1e-5)
    y = g_ref[...][None, None, :] * y + b_ref[...][None, None, :]
    y = jnp.maximum(y, 0.0)
    out_ref[...] = y * dis_ref[0, 0][None, :, None]


def _stage1(x_pad, w_t, b_t, gamma, beta, dis):
    grid = (NPAD // NB1,)
    return pl.pallas_call(
        _stage1_body,
        grid=grid,
        in_specs=[
            pl.BlockSpec((T, NB1, C), lambda i: (0, i, 0)),
            pl.BlockSpec((C, C, 3), lambda i: (0, 0, 0)),
            pl.BlockSpec((C,), lambda i: (0,)),
            pl.BlockSpec((C,), lambda i: (0,)),
            pl.BlockSpec((C,), lambda i: (0,)),
            pl.BlockSpec((1, 1, NB1), lambda i: (i, 0, 0)),
        ],
        out_specs=pl.BlockSpec((T, NB1, C), lambda i: (0, i, 0)),
        out_shape=jax.ShapeDtypeStruct((T, NPAD, C), jnp.float32),
    )(x_pad, w_t, b_t, gamma, beta, dis.reshape(NPAD // NB1, 1, NB1))


def _main_body(table_hbm, epack_hbm, zeros_hbm, out_hbm,
               ebuf0, ebuf1, idx0, idx1, rows0, rows1, acc, sem0, sem1):
    cid = lax.axis_index("c")
    sid = lax.axis_index("s")
    wid = cid * NS + sid

    ebufs = (ebuf0, ebuf1)
    idxs = (idx0, idx1)
    rows = (rows0, rows1)
    sems = (sem0, sem1)

    def acc_zero():
        def z(i, _):
            pltpu.sync_copy(zeros_hbm,
                            acc.at[pl.ds(sid * ROWS_PT + i * B, B)])
            return _
        lax.fori_loop(0, ROWS_PT // B, z, 0)

    acc_zero()
    plsc.subcore_barrier()

    def issue(j, buf, toff):
        pltpu.sync_copy(epack_hbm.at[wid, j], ebufs[buf])

        def grp(g, _):
            sl = pl.ds(g * L, L)
            idxs[buf][sl] = ebufs[buf][0, sl] + toff
            return _
        lax.fori_loop(0, B // L, grp, 0)
        pltpu.make_async_copy(table_hbm.at[idxs[buf]], rows[buf],
                              sems[buf]).start()

    def finish(buf):
        pltpu.make_async_copy(table_hbm.at[idxs[buf]], rows[buf],
                              sems[buf]).wait()

        def scale(b, _):
            wi = plsc.load_gather(ebufs[buf].at[2],
                                  [jnp.full((L,), b, jnp.int32)])
            nb = plsc.bitcast(wi, jnp.float32)
            for k in range(C // L):
                sl = pl.ds(k * L, L)
                rows[buf][b, sl] = rows[buf][b, sl] * nb
            return _
        lax.fori_loop(0, B, scale, 0)
        pltpu.sync_copy(rows[buf], acc.at[ebufs[buf].at[1]], add=True)

    def t_step(t, _):
        toff = t * NPAD

        issue(0, 0, toff)

        def pair(p, _):
            issue(2 * p + 1, 1, toff)
            finish(0)
            issue(2 * p + 2, 0, toff)
            finish(1)
            return _
        lax.fori_loop(0, (NCH - 1) // 2, pair, 0)
        finish(0)
        plsc.subcore_barrier()

        pltpu.sync_copy(acc.at[pl.ds(sid * ROWS_PT, ROWS_PT)],
                        out_hbm.at[cid, t, pl.ds(sid * ROWS_PT, ROWS_PT)])
        acc_zero()
        plsc.subcore_barrier()
        return _
    lax.fori_loop(0, T, t_step, 0)


def _main(table, epack, zeros):
    mesh = plsc.VectorSubcoreMesh(
        core_axis_name="c", subcore_axis_name="s", num_cores=NC, num_subcores=NS)
    return pl.kernel(
        _main_body,
        out_type=jax.ShapeDtypeStruct((NC, T, NPAD, C), jnp.float32),
        mesh=mesh,
        compiler_params=pltpu.CompilerParams(needs_layout_passes=False),
        scratch_types=[
            pltpu.VMEM((3, B), jnp.int32),
            pltpu.VMEM((3, B), jnp.int32),
            pltpu.VMEM((B,), jnp.int32),
            pltpu.VMEM((B,), jnp.int32),
            pltpu.VMEM((B, C), jnp.float32),
            pltpu.VMEM((B, C), jnp.float32),
            pltpu.VMEM_SHARED((NPAD, C), jnp.float32),
            pltpu.SemaphoreType.DMA,
            pltpu.SemaphoreType.DMA,
        ],
    )(table, epack, zeros)


NB6 = 512


def _out_body(agg_ref, xc_ref, dis_ref, wg_ref, bg_ref, out_ref):
    a = agg_ref[0, 0] + agg_ref[1, 0]
    d = dis_ref[0, 0]
    xcb = xc_ref[0]
    full = (a + xcb) * d[:, None]
    hp = jax.lax.Precision.HIGHEST
    z = lax.dot_general(full, wg_ref[...], (((1,), (1,)), ((), ())),
                        precision=hp, preferred_element_type=jnp.float32)
    z = z + bg_ref[...][None, :]
    z = jnp.maximum(z, 0.0)
    out_ref[0] = jnp.minimum(z, HI)


def _out(agg, xc, dis, w_g, b_g):
    grid = (T, NPAD // NB6)
    return pl.pallas_call(
        _out_body,
        grid=grid,
        in_specs=[
            pl.BlockSpec((NC, 1, NB6, C), lambda t, i: (0, t, i, 0)),
            pl.BlockSpec((1, NB6, C), lambda t, i: (t, i, 0)),
            pl.BlockSpec((1, 1, NB6), lambda t, i: (i, 0, 0)),
            pl.BlockSpec((C, C), lambda t, i: (0, 0)),
            pl.BlockSpec((C,), lambda t, i: (0,)),
        ],
        out_specs=pl.BlockSpec((1, NB6, C), lambda t, i: (t, i, 0)),
        out_shape=jax.ShapeDtypeStruct((T, NPAD, C), jnp.float32),
    )(agg, xc, dis.reshape(NPAD // NB6, 1, NB6), w_g, b_g)


def kernel(x, edge_index, edge_weight, w_t, b_t, gamma, beta, w_g, b_g):
    x_pad = jnp.pad(x, ((0, 0), (0, NPAD - N), (0, 0)))
    src = edge_index[0]
    dst = edge_index[1]
    ew_n = _softmax(edge_weight)
    src_p = jnp.pad(src, (0, EPAD - E))
    dst_p = jnp.pad(dst, (0, EPAD - E))
    ewn_p = jnp.pad(ew_n, (0, EPAD - E))

    dst3 = dst_p.reshape(NW, NCH, B)
    degp = _deg(dst3, ewn_p)
    dis = _dis(degp)
    xc = _stage1(x_pad, w_t, b_t, gamma, beta, dis)

    table = xc.reshape(T * NPAD, C)
    src3 = src_p.reshape(NW, NCH, B)
    ew3 = lax.bitcast_convert_type(ewn_p, jnp.int32).reshape(NW, NCH, B)
    epack = jnp.stack([src3, dst3, ew3], axis=2)
    zeros = jnp.zeros((B, C), jnp.float32)
    agg = _main(table, epack, zeros)

    out = _out(agg, xc, dis, w_g, b_g)
    return out[:, :N, :]


# speedup vs baseline: 6.6392x; 1.0248x over previous
import functools

import jax
import jax.numpy as jnp
from jax import lax
from jax.experimental import pallas as pl
from jax.experimental.pallas import tpu as pltpu
from jax.experimental.pallas import tpu_sc as plsc

T = 12
N = 10000
E = 320000
C = 128

NPAD = 10240
NC, NS, L = 2, 16, 16
NW = NC * NS
B = 128
EPT = 10112
NCH = EPT // B
EPAD = EPT * NW
ROWS_PT = NPAD // NS
HI = 10.0


def _softmax_body(ew_ref, out_ref):
    w = ew_ref[...]
    m = jnp.max(w)
    e = jnp.exp(w - m)
    out_ref[...] = e / jnp.sum(e)


def _softmax(ew):
    ew2 = ew.reshape(E // C, C)
    out = pl.pallas_call(
        _softmax_body,
        out_shape=jax.ShapeDtypeStruct((E // C, C), jnp.float32),
    )(ew2)
    return out.reshape(E)


DW = 16


def _deg_body(dst3_hbm, ewf_hbm, out_hbm, dst2, ewc, rows, acc, sem):
    cid = lax.axis_index("c")
    sid = lax.axis_index("s")
    wid = cid * NS + sid

    pltpu.sync_copy(dst3_hbm.at[wid], dst2)
    pltpu.sync_copy(ewf_hbm.at[pl.ds(wid * EPT, EPT)], ewc)

    if True:
        def zrow(i, _):
            rows[i] = jnp.zeros((DW,), jnp.float32)
            return _
        lax.fori_loop(0, B, zrow, 0)

        def z(i, _):
            pltpu.sync_copy(rows, acc.at[pl.ds(sid * ROWS_PT + i * B, B)])
            return _
        lax.fori_loop(0, ROWS_PT // B, z, 0)
        plsc.subcore_barrier()

        def chunk(j, _):
            def fill(b, _):
                wv = plsc.load_gather(
                    ewc, [jnp.full((L,), j * B + b, jnp.int32)])
                rows[b] = wv
                return _
            lax.fori_loop(0, B, fill, 0)
            pltpu.sync_copy(rows, acc.at[dst2.at[j]], add=True)
            return _
        lax.fori_loop(0, NCH, chunk, 0)
        plsc.subcore_barrier()
        pltpu.sync_copy(acc.at[pl.ds(sid * ROWS_PT, ROWS_PT)],
                        out_hbm.at[cid, pl.ds(sid * ROWS_PT, ROWS_PT)])


def _deg(dst3, ewn_p):
    mesh = plsc.VectorSubcoreMesh(
        core_axis_name="c", subcore_axis_name="s", num_cores=NC, num_subcores=NS)
    return pl.kernel(
        _deg_body,
        out_type=jax.ShapeDtypeStruct((NC, NPAD, DW), jnp.float32),
        mesh=mesh,
        compiler_params=pltpu.CompilerParams(needs_layout_passes=False),
        scratch_types=[
            pltpu.VMEM((NCH, B), jnp.int32),
            pltpu.VMEM((EPT,), jnp.float32),
            pltpu.VMEM((B, DW), jnp.float32),
            pltpu.VMEM_SHARED((NPAD, DW), jnp.float32),
            pltpu.SemaphoreType.DMA,
        ],
    )(dst3, ewn_p)


def _dis_body(degp_ref, out_ref):
    deg = degp_ref[0, :, 0] + degp_ref[1, :, 0]
    out_ref[...] = lax.rsqrt(deg.reshape(NPAD // C, C) + 1.0)


def _dis(degp):
    out = pl.pallas_call(
        _dis_body,
        out_shape=jax.ShapeDtypeStruct((NPAD // C, C), jnp.float32),
    )(degp)
    return out.reshape(NPAD)


NB1 = 256


def _stage1_body(x_ref, wt_ref, bt_ref, g_ref, b_ref, dis_ref, out_ref):
    xb = x_ref[...]
    W = wt_ref[...]
    xf = xb.reshape(T * NB1, C)
    dn = (((1,), (1,)), ((), ()))
    hp = jax.lax.Precision.HIGHEST
    y1 = lax.dot_general(xf, W[:, :, 1], dn, precision=hp,
                         preferred_element_type=jnp.float32).reshape(T, NB1, C)
    z0 = lax.dot_general(xf, W[:, :, 0], dn, precision=hp,
                         preferred_element_type=jnp.float32).reshape(T, NB1, C)
    z2 = lax.dot_general(xf, W[:, :, 2], dn, precision=hp,
                         preferred_element_type=jnp.float32).reshape(T, NB1, C)
    zpad = jnp.zeros((1, NB1, C), jnp.float32)
    y = y1 + jnp.concatenate([zpad, z0[:-1]], 0) \
           + jnp.concatenate([z2[1:], zpad], 0)
    y = y + bt_ref[...][None, None, :]
    y = jnp.clip(y, -HI, HI)
    mean = jnp.mean(y, axis=0, keepdims=True)
    var = jnp.mean((y - mean) ** 2, axis=0, keepdims=True)
    y = (y - mean) * lax.rsqrt(var + 1e-5)
    y = g_ref[...][None, None, :] * y + b_ref[...][None, None, :]
    y = jnp.maximum(y, 0.0)
    y = y * dis_ref[0, 0][None, :, None]
    out_ref[...] = y


def _stage1(x_pad, w_t, b_t, gamma, beta, dis):
    grid = (NPAD // NB1,)
    return pl.pallas_call(
        _stage1_body,
        grid=grid,
        in_specs=[
            pl.BlockSpec((T, NB1, C), lambda i: (0, i, 0)),
            pl.BlockSpec((C, C, 3), lambda i: (0, 0, 0)),
            pl.BlockSpec((C,), lambda i: (0,)),
            pl.BlockSpec((C,), lambda i: (0,)),
            pl.BlockSpec((C,), lambda i: (0,)),
            pl.BlockSpec((1, 1, NB1), lambda i: (i, 0, 0)),
        ],
        out_specs=pl.BlockSpec((T, NB1, C), lambda i: (0, i, 0)),
        out_shape=jax.ShapeDtypeStruct((T, NPAD, C), jnp.float32),
    )(x_pad, w_t, b_t, gamma, beta, dis.reshape(NPAD // NB1, 1, NB1))


ZB = 32


def _main_body(table_hbm, epack_hbm, out_hbm,
               ebuf0, ebuf1, idx0, idx1, rg0, rg1, zbuf, acc, sem0, sem1):
    cid = lax.axis_index("c")
    sid = lax.axis_index("s")
    wid = cid * NS + sid

    ebufs = (ebuf0, ebuf1)
    idxs = (idx0, idx1)
    rgs = (rg0, rg1)
    sems = (sem0, sem1)

    def zbuf_zero():
        def zrow(i, _):
            for k in range(C // L):
                zbuf[i, pl.ds(k * L, L)] = jnp.zeros((L,), jnp.float32)
            return _
        lax.fori_loop(0, ZB, zrow, 0)

    def acc_zero():
        def z(i, _):
            pltpu.sync_copy(zbuf,
                            acc.at[pl.ds(sid * ROWS_PT + i * ZB, ZB)])
            return _
        lax.fori_loop(0, ROWS_PT // ZB, z, 0)

    zbuf_zero()
    acc_zero()
    plsc.subcore_barrier()

    def issue(j, buf, toff):
        pltpu.sync_copy(epack_hbm.at[wid, j], ebufs[buf])

        for g in range(B // L):
            sl = pl.ds(g * L, L)
            idxs[buf][sl] = ebufs[buf][0, sl] + toff
        pltpu.make_async_copy(table_hbm.at[idxs[buf]], rgs[buf],
                              sems[buf]).start()

    def finish(buf):
        pltpu.make_async_copy(table_hbm.at[idxs[buf]], rgs[buf],
                              sems[buf]).wait()

        def scale(q, _):
            for u in range(4):
                b = q * 4 + u
                wi = plsc.load_gather(ebufs[buf].at[2],
                                      [jnp.full((L,), b, jnp.int32)])
                nb = plsc.bitcast(wi, jnp.float32)
                for k in range(C // L):
                    sl = pl.ds(k * L, L)
                    rgs[buf][b, sl] = rgs[buf][b, sl] * nb
            return _
        lax.fori_loop(0, B // 4, scale, 0)
        pltpu.sync_copy(rgs[buf], acc.at[ebufs[buf].at[1]], add=True)

    def t_step(t, _):
        toff = t * NPAD

        issue(0, 0, toff)

        def pair(p, _):
            issue(2 * p + 1, 1, toff)
            finish(0)
            issue(2 * p + 2, 0, toff)
            finish(1)
            return _
        lax.fori_loop(0, (NCH - 1) // 2, pair, 0)
        finish(0)
        plsc.subcore_barrier()

        pltpu.sync_copy(acc.at[pl.ds(sid * ROWS_PT, ROWS_PT)],
                        out_hbm.at[cid, t, pl.ds(sid * ROWS_PT, ROWS_PT)])
        acc_zero()
        plsc.subcore_barrier()
        return _
    lax.fori_loop(0, T, t_step, 0)


def _main(table, epack):
    mesh = plsc.VectorSubcoreMesh(
        core_axis_name="c", subcore_axis_name="s", num_cores=NC, num_subcores=NS)
    return pl.kernel(
        _main_body,
        out_type=jax.ShapeDtypeStruct((NC, T, NPAD, C), jnp.float32),
        mesh=mesh,
        compiler_params=pltpu.CompilerParams(needs_layout_passes=False),
        scratch_types=[
            pltpu.VMEM((3, B), jnp.int32),
            pltpu.VMEM((3, B), jnp.int32),
            pltpu.VMEM((B,), jnp.int32),
            pltpu.VMEM((B,), jnp.int32),
            pltpu.VMEM((B, C), jnp.float32),
            pltpu.VMEM((B, C), jnp.float32),
            pltpu.VMEM((ZB, C), jnp.float32),
            pltpu.VMEM_SHARED((NPAD, C), jnp.float32),
            pltpu.SemaphoreType.DMA,
            pltpu.SemaphoreType.DMA,
        ],
    )(table, epack)


NB6 = 512


def _out_body(agg_ref, xc_ref, dis_ref, wg_ref, bg_ref, out_ref):
    a = agg_ref[0, 0] + agg_ref[1, 0]
    d = dis_ref[0, 0]
    xcb = xc_ref[0]
    full = (a + xcb) * d[:, None]
    hp = jax.lax.Precision.HIGHEST
    z = lax.dot_general(full, wg_ref[...], (((1,), (1,)), ((), ())),
                        precision=hp, preferred_element_type=jnp.float32)
    z = z + bg_ref[...][None, :]
    z = jnp.maximum(z, 0.0)
    out_ref[0] = jnp.minimum(z, HI)


def _out(agg, xc, dis, w_g, b_g):
    grid = (T, NPAD // NB6)
    return pl.pallas_call(
        _out_body,
        grid=grid,
        in_specs=[
            pl.BlockSpec((NC, 1, NB6, C), lambda t, i: (0, t, i, 0)),
            pl.BlockSpec((1, NB6, C), lambda t, i: (t, i, 0)),
            pl.BlockSpec((1, 1, NB6), lambda t, i: (i, 0, 0)),
            pl.BlockSpec((C, C), lambda t, i: (0, 0)),
            pl.BlockSpec((C,), lambda t, i: (0,)),
        ],
        out_specs=pl.BlockSpec((1, NB6, C), lambda t, i: (t, i, 0)),
        out_shape=jax.ShapeDtypeStruct((T, NPAD, C), jnp.float32),
    )(agg, xc, dis.reshape(NPAD // NB6, 1, NB6), w_g, b_g)


def kernel(x, edge_index, edge_weight, w_t, b_t, gamma, beta, w_g, b_g):
    x_pad = jnp.pad(x, ((0, 0), (0, NPAD - N), (0, 0)))
    src = edge_index[0]
    dst = edge_index[1]
    ew_n = _softmax(edge_weight)
    src_p = jnp.pad(src, (0, EPAD - E))
    dst_p = jnp.pad(dst, (0, EPAD - E))
    ewn_p = jnp.pad(ew_n, (0, EPAD - E))

    dst3 = dst_p.reshape(NW, NCH, B)
    degp = _deg(dst3, ewn_p)
    dis = _dis(degp)
    xc = _stage1(x_pad, w_t, b_t, gamma, beta, dis)

    table = xc.reshape(T * NPAD, C)
    src3 = src_p.reshape(NW, NCH, B)
    ew3 = lax.bitcast_convert_type(ewn_p, jnp.int32).reshape(NW, NCH, B)
    epack = jnp.stack([src3, dst3, ew3], axis=2)
    agg = _main(table, epack)

    out = _out(agg, xc, dis, w_g, b_g)
    return out[:, :N, :]
